# baseline (speedup 1.0000x reference)
import functools

import jax
import jax.numpy as jnp
from jax import lax
from jax.experimental import pallas as pl
from jax.experimental.pallas import tpu as pltpu
from jax.experimental.pallas import tpu_sc as plsc

_D = 64
_PAD = 0
_NC = 2
_NS = 16
_NW = _NC * _NS
_CHUNK = 128


def _embed_lookup(idx3, table, n_chunks):
  mesh = plsc.VectorSubcoreMesh(core_axis_name="c", subcore_axis_name="s")

  @functools.partial(
      pl.kernel,
      out_type=jax.ShapeDtypeStruct((_NW, n_chunks, _CHUNK, _D), jnp.float32),
      mesh=mesh,
      compiler_params=pltpu.CompilerParams(
          needs_layout_passes=False, use_tc_tiling_on_sc=False),
      scratch_types=[
          pltpu.VMEM((n_chunks, _CHUNK), jnp.int32),
          pltpu.VMEM((_CHUNK, _D), jnp.float32),
          pltpu.VMEM((_CHUNK, _D), jnp.float32),
          pltpu.VMEM((16,), jnp.int32),
          pltpu.SemaphoreType.DMA,
          pltpu.SemaphoreType.DMA,
      ],
  )
  def run(idx_hbm, table_hbm, out_hbm, idx_v, rows0, rows1, flag_v, gsem0,
          gsem1):
    wid = lax.axis_index("s") * _NC + lax.axis_index("c")
    pltpu.sync_copy(idx_hbm.at[wid], idx_v)

    rows = (rows0, rows1)
    gsems = (gsem0, gsem1)

    def fire(j, s):
      pltpu.make_async_copy(table_hbm.at[idx_v.at[j]], rows[s], gsems[s]).start()

    def handle(j, s):
      pltpu.make_async_copy(table_hbm.at[idx_v.at[j]], rows[s], gsems[s]).wait()
      idx_row = idx_v.at[j]
      msk_acc = idx_row[pl.ds(0, 16)] == _PAD
      for g in range(1, _CHUNK // 16):
        msk_acc = msk_acc | (idx_row[pl.ds(16 * g, 16)] == _PAD)
      flag_v[...] = jnp.zeros((16,), jnp.int32)
      plsc.store_scatter(flag_v.at[...], [jnp.zeros((16,), jnp.int32)],
                         jnp.ones((16,), jnp.int32), mask=msk_acc)
      nz = flag_v[...][0]

      @pl.when(nz != 0)
      def _fixup():
        zero16 = jnp.zeros((16,), jnp.float32)
        for g in range(_CHUNK // 16):
          v = idx_row[pl.ds(16 * g, 16)]
          msk = v == _PAD
          rowv = 16 * g + lax.iota(jnp.int32, 16)

          def cbody(c, carry):
            colv = jnp.zeros((16,), jnp.int32) + c
            plsc.store_scatter(rows[s].at[...], [rowv, colv], zero16, mask=msk)
            return carry

          lax.fori_loop(0, _D, cbody, 0)

      pltpu.sync_copy(rows[s], out_hbm.at[wid, j])

    fire(0, 0)

    def body2(t, carry):
      j0 = 2 * t
      fire(j0 + 1, 1)
      handle(j0, 0)

      @pl.when(j0 + 2 < n_chunks)
      def _next():
        fire(j0 + 2, 0)

      handle(j0 + 1, 1)
      return carry

    lax.fori_loop(0, n_chunks // 2, body2, 0)

  return run(idx3, table)


def kernel(text, text_lengths, embedding_weight):
  del text_lengths
  b, s = text.shape
  total = b * s
  assert total % (_NW * _CHUNK) == 0
  n_chunks = total // (_NW * _CHUNK)
  idx3 = text.reshape(_NW, n_chunks, _CHUNK).astype(jnp.int32)
  out = _embed_lookup(idx3, embedding_weight, n_chunks)
  return out.reshape(b, s, _D)


# speedup vs baseline: 1.0215x; 1.0215x over previous
import functools

import jax
import jax.numpy as jnp
from jax import lax
from jax.experimental import pallas as pl
from jax.experimental.pallas import tpu as pltpu
from jax.experimental.pallas import tpu_sc as plsc

_D = 64
_PAD = 0
_NC = 2
_NS = 16
_NW = _NC * _NS
_CHUNK = 256
_NBUF = 4
_LOOK = 2


def _embed_lookup(idx3, table, n_chunks):
  mesh = plsc.VectorSubcoreMesh(core_axis_name="c", subcore_axis_name="s")

  @functools.partial(
      pl.kernel,
      out_type=jax.ShapeDtypeStruct((_NW, n_chunks, _CHUNK, _D), jnp.float32),
      mesh=mesh,
      compiler_params=pltpu.CompilerParams(
          needs_layout_passes=False, use_tc_tiling_on_sc=False),
      scratch_types=[
          pltpu.VMEM((n_chunks, _CHUNK), jnp.int32),
          [pltpu.VMEM((_CHUNK, _D), jnp.float32) for _ in range(_NBUF)],
          pltpu.VMEM((16,), jnp.int32),
          [pltpu.SemaphoreType.DMA for _ in range(_NBUF)],
          [pltpu.SemaphoreType.DMA for _ in range(_NBUF)],
      ],
  )
  def run(idx_hbm, table_hbm, out_hbm, idx_v, rows, flag_v, gsems, osems):
    wid = lax.axis_index("s") * _NC + lax.axis_index("c")
    pltpu.sync_copy(idx_hbm.at[wid], idx_v)

    def fire(j, s):
      pltpu.make_async_copy(table_hbm.at[idx_v.at[j]], rows[s], gsems[s]).start()

    def out_start(j, s):
      pltpu.make_async_copy(rows[s], out_hbm.at[wid, j], osems[s]).start()

    def out_wait(j, s):
      pltpu.make_async_copy(rows[s], out_hbm.at[wid, j], osems[s]).wait()

    def handle(j, s):
      pltpu.make_async_copy(table_hbm.at[idx_v.at[j]], rows[s], gsems[s]).wait()
      idx_row = idx_v.at[j]
      msk_acc = idx_row[pl.ds(0, 16)] == _PAD
      for g in range(1, _CHUNK // 16):
        msk_acc = msk_acc | (idx_row[pl.ds(16 * g, 16)] == _PAD)
      flag_v[...] = jnp.zeros((16,), jnp.int32)
      plsc.store_scatter(flag_v.at[...], [jnp.zeros((16,), jnp.int32)],
                         jnp.ones((16,), jnp.int32), mask=msk_acc)
      nz = flag_v[...][0]

      @pl.when(nz != 0)
      def _fixup():
        zero16 = jnp.zeros((16,), jnp.float32)
        for g in range(_CHUNK // 16):
          v = idx_row[pl.ds(16 * g, 16)]
          msk = v == _PAD
          rowv = 16 * g + lax.iota(jnp.int32, 16)

          def cbody(c, carry):
            colv = jnp.zeros((16,), jnp.int32) + c
            plsc.store_scatter(rows[s].at[...], [rowv, colv], zero16, mask=msk)
            return carry

          lax.fori_loop(0, _D, cbody, 0)

      out_start(j, s)

    for j in range(_LOOK):
      fire(j, j % _NBUF)

    def body4(t, carry):
      for b in range(_NBUF):
        j = _NBUF * t + b
        handle(j, b)
        g = j + _LOOK
        s2 = (b + _LOOK) % _NBUF

        @pl.when(g < n_chunks)
        def _next():
          @pl.when(g >= _NBUF)
          def _drain():
            out_wait(g - _NBUF, s2)

          fire(g, s2)

      return carry

    lax.fori_loop(0, n_chunks // _NBUF, body4, 0)

    for b in range(_NBUF):
      out_wait(n_chunks - _NBUF + b, b)

  return run(idx3, table)


def kernel(text, text_lengths, embedding_weight):
  del text_lengths
  b, s = text.shape
  total = b * s
  assert total % (_NW * _CHUNK * _NBUF) == 0
  n_chunks = total // (_NW * _CHUNK)
  idx3 = text.reshape(_NW, n_chunks, _CHUNK).astype(jnp.int32)
  out = _embed_lookup(idx3, embedding_weight, n_chunks)
  return out.reshape(b, s, _D)
